# baseline (speedup 1.0000x reference)
import functools

import jax
import jax.numpy as jnp
from jax import lax
from jax.experimental import pallas as pl
from jax.experimental.pallas import tpu as pltpu
from jax.experimental.pallas import tpu_sc as plsc

N_NODES = 10000
D = 128
E_POS = 320000
E_NEG = 320000
E_TOT = E_POS + E_NEG
E_HALF = E_TOT // 2

NC = 2
NS = 16
L = 16

CHUNK = 64
CROWS = CHUNK * L // 128
E_PER_W = 312 * CHUNK
NCHUNK_BASE = 312
NCHUNK_LAST = 320
_ROWS_H = E_HALF * L // 128

_sc_mesh = plsc.VectorSubcoreMesh(core_axis_name="c", subcore_axis_name="s")


@functools.partial(
    pl.kernel,
    out_type=(
        jax.ShapeDtypeStruct((_ROWS_H, 128), jnp.float32),
        jax.ShapeDtypeStruct((_ROWS_H, 128), jnp.float32),
    ),
    mesh=_sc_mesh,
    scratch_types=[
        pltpu.VMEM_SHARED((N_NODES, D), jnp.float32),
        pltpu.VMEM((CHUNK,), jnp.int32),
        pltpu.VMEM((CHUNK,), jnp.int32),
        pltpu.VMEM((CHUNK,), jnp.int32),
        pltpu.VMEM((CHUNK,), jnp.int32),
        pltpu.VMEM((CHUNK, D), jnp.float32),
        pltpu.VMEM((CHUNK, D), jnp.float32),
        pltpu.VMEM((CHUNK, D), jnp.float32),
        pltpu.VMEM((CHUNK, D), jnp.float32),
        pltpu.VMEM((CROWS, 128), jnp.float32),
        pltpu.VMEM((CROWS, 128), jnp.float32),
        pltpu.SemaphoreType.DMA,
        pltpu.SemaphoreType.DMA,
        pltpu.SemaphoreType.DMA,
        pltpu.SemaphoreType.DMA,
        pltpu.SemaphoreType.DMA,
        pltpu.SemaphoreType.DMA,
        pltpu.SemaphoreType.DMA,
        pltpu.SemaphoreType.DMA,
    ],
)
def _edge_dot_sc(table_hbm, psrc_hbm, pdst_hbm, nsrc_hbm, ndst_hbm,
                 out0_hbm, out1_hbm,
                 table_sh, sidx_a, didx_a, sidx_b, didx_b,
                 srow_a, drow_a, srow_b, drow_b,
                 part_a, part_b,
                 sem_ia, sem_ib, sem_sa, sem_da, sem_sb, sem_db,
                 sem_oa, sem_ob):
    cid = lax.axis_index("c")
    sid = lax.axis_index("s")
    hbase = sid * E_PER_W
    rbase = sid * (E_PER_W * L // 128)
    nchunk = jnp.where(sid == NS - 1, NCHUNK_LAST, NCHUNK_BASE)

    @pl.when(sid == 0)
    def _stage_table():
        pltpu.sync_copy(table_hbm, table_sh)

    plsc.subcore_barrier()

    def compute(srow, drow, part):
        @plsc.parallel_loop(0, CROWS, unroll=2)
        def _row(r):
            for j in range(128 // L):
                e = r * (128 // L) + j
                acc = srow[e, pl.ds(0, L)] * drow[e, pl.ds(0, L)]
                for k in range(1, D // L):
                    acc = acc + (srow[e, pl.ds(k * L, L)]
                                 * drow[e, pl.ds(k * L, L)])
                part[r, pl.ds(j * L, L)] = acc

    def run(esrc_hbm, edst_hbm, out_hbm):
        def issue_idx(g, sidx, didx, sem_i):
            off = hbase + g * CHUNK
            pltpu.async_copy(esrc_hbm.at[pl.ds(off, CHUNK)], sidx, sem_i)
            pltpu.async_copy(edst_hbm.at[pl.ds(off, CHUNK)], didx, sem_i)

        def wait_idx(sidx, didx, sem_i):
            pltpu.make_async_copy(
                esrc_hbm.at[pl.ds(hbase, CHUNK)], sidx, sem_i).wait()
            pltpu.make_async_copy(
                edst_hbm.at[pl.ds(hbase, CHUNK)], didx, sem_i).wait()

        def issue_rows_a(sidx, didx, srow, drow, sem_s, sem_d):
            pltpu.async_copy(table_sh.at[sidx], srow, sem_s)
            pltpu.async_copy(table_sh.at[didx], drow, sem_d)

        def wait_rows_a(sidx, didx, srow, drow, sem_s, sem_d):
            pltpu.make_async_copy(table_sh.at[sidx], srow, sem_s).wait()
            pltpu.make_async_copy(table_sh.at[didx], drow, sem_d).wait()

        def issue_rows_b(sidx, didx, srow, drow, sem_s, sem_d):
            pltpu.async_copy(table_hbm.at[sidx], srow, sem_s)
            pltpu.async_copy(table_hbm.at[didx], drow, sem_d)

        def wait_rows_b(sidx, didx, srow, drow, sem_s, sem_d):
            pltpu.make_async_copy(table_hbm.at[sidx], srow, sem_s).wait()
            pltpu.make_async_copy(table_hbm.at[didx], drow, sem_d).wait()

        def wait_out(part, sem_o):
            pltpu.make_async_copy(
                part, out_hbm.at[pl.ds(rbase, CROWS)], sem_o).wait()

        issue_idx(0, sidx_a, didx_a, sem_ia)
        issue_idx(1, sidx_b, didx_b, sem_ib)
        wait_idx(sidx_a, didx_a, sem_ia)
        issue_rows_a(sidx_a, didx_a, srow_a, drow_a, sem_sa, sem_da)

        def pair_body(i, carry):
            ga = 2 * i
            gb = ga + 1
            wait_idx(sidx_b, didx_b, sem_ib)
            issue_rows_b(sidx_b, didx_b, srow_b, drow_b, sem_sb, sem_db)

            wait_rows_a(sidx_a, didx_a, srow_a, drow_a, sem_sa, sem_da)

            @pl.when(ga + 2 < nchunk)
            def _idx_a():
                issue_idx(ga + 2, sidx_a, didx_a, sem_ia)

            @pl.when(i > 0)
            def _drain_a():
                wait_out(part_a, sem_oa)

            compute(srow_a, drow_a, part_a)
            pltpu.async_copy(
                part_a, out_hbm.at[pl.ds(rbase + ga * CROWS, CROWS)], sem_oa)

            wait_rows_b(sidx_b, didx_b, srow_b, drow_b, sem_sb, sem_db)

            @pl.when(gb + 2 < nchunk)
            def _idx_b():
                issue_idx(gb + 2, sidx_b, didx_b, sem_ib)

            @pl.when(i > 0)
            def _drain_b():
                wait_out(part_b, sem_ob)

            compute(srow_b, drow_b, part_b)
            pltpu.async_copy(
                part_b, out_hbm.at[pl.ds(rbase + gb * CROWS, CROWS)], sem_ob)

            @pl.when(ga + 2 < nchunk)
            def _rows_a():
                wait_idx(sidx_a, didx_a, sem_ia)
                issue_rows_a(sidx_a, didx_a, srow_a, drow_a, sem_sa, sem_da)

            return carry

        lax.fori_loop(0, nchunk // 2, pair_body, 0)
        wait_out(part_a, sem_oa)
        wait_out(part_b, sem_ob)

    @pl.when(cid == 0)
    def _core0():
        run(psrc_hbm, pdst_hbm, out0_hbm)

    @pl.when(cid == 1)
    def _core1():
        run(nsrc_hbm, ndst_hbm, out1_hbm)


_BLK = 5000
_NBLK = _ROWS_H // _BLK


def _loss_body(p_pos_ref, p_neg_ref, out_ref):
    pid = pl.program_id(0)
    gj = lax.broadcasted_iota(jnp.int32, (128, 8), 0) // L
    gg = lax.broadcasted_iota(jnp.int32, (128, 8), 1)
    m = (gj == gg).astype(jnp.float32)
    s0 = jnp.dot(p_pos_ref[...], m,
                 preferred_element_type=jnp.float32)
    s1 = jnp.dot(p_neg_ref[...], m,
                 preferred_element_type=jnp.float32)
    l0 = jnp.maximum(-s0, 0.0) + jnp.log1p(jnp.exp(-jnp.abs(s0)))
    l1 = jnp.maximum(s1, 0.0) + jnp.log1p(jnp.exp(-jnp.abs(s1)))
    bsum = jnp.sum(l0) + jnp.sum(l1)

    @pl.when(pid == 0)
    def _init():
        out_ref[0, 0] = 0.0

    out_ref[0, 0] += bsum

    @pl.when(pid == pl.num_programs(0) - 1)
    def _fini():
        out_ref[0, 0] = out_ref[0, 0] / E_TOT


_loss_tc = pl.pallas_call(
    _loss_body,
    grid=(_NBLK,),
    in_specs=[
        pl.BlockSpec((_BLK, 128), lambda i: (i, 0)),
        pl.BlockSpec((_BLK, 128), lambda i: (i, 0)),
    ],
    out_specs=pl.BlockSpec(
        (1, 1), lambda i: (0, 0), memory_space=pltpu.SMEM),
    out_shape=jax.ShapeDtypeStruct((1, 1), jnp.float32),
)


def kernel(block_outputs, pos_edge_index, neg_edge_index):
    pe = pos_edge_index.astype(jnp.int32)
    ne = neg_edge_index.astype(jnp.int32)
    p_pos, p_neg = _edge_dot_sc(block_outputs, pe[0], pe[1], ne[0], ne[1])
    loss = _loss_tc(p_pos, p_neg)
    return loss[0, 0]


# speedup vs baseline: 1.0133x; 1.0133x over previous
import functools

import jax
import jax.numpy as jnp
from jax import lax
from jax.experimental import pallas as pl
from jax.experimental.pallas import tpu as pltpu
from jax.experimental.pallas import tpu_sc as plsc

N_NODES = 10000
D = 128
E_POS = 320000
E_NEG = 320000
E_TOT = E_POS + E_NEG
E_HALF = E_TOT // 2

NC = 2
NS = 16
L = 16

CHUNK = 64
CROWS = CHUNK * L // 128
E_PER_W = 312 * CHUNK
NCHUNK_BASE = 312
NCHUNK_LAST = 320
_ROWS_H = E_HALF * L // 128

_sc_mesh = plsc.VectorSubcoreMesh(core_axis_name="c", subcore_axis_name="s")


@functools.partial(
    pl.kernel,
    out_type=(
        jax.ShapeDtypeStruct((_ROWS_H, 128), jnp.float32),
        jax.ShapeDtypeStruct((_ROWS_H, 128), jnp.float32),
    ),
    mesh=_sc_mesh,
    scratch_types=[
        pltpu.VMEM_SHARED((N_NODES, D), jnp.float32),
        pltpu.VMEM((CHUNK,), jnp.int32),
        pltpu.VMEM((CHUNK,), jnp.int32),
        pltpu.VMEM((CHUNK,), jnp.int32),
        pltpu.VMEM((CHUNK,), jnp.int32),
        pltpu.VMEM((CHUNK, D), jnp.float32),
        pltpu.VMEM((CHUNK, D), jnp.float32),
        pltpu.VMEM((CHUNK, D), jnp.float32),
        pltpu.VMEM((CHUNK, D), jnp.float32),
        pltpu.VMEM((CROWS, 128), jnp.float32),
        pltpu.VMEM((CROWS, 128), jnp.float32),
        pltpu.SemaphoreType.DMA,
        pltpu.SemaphoreType.DMA,
        pltpu.SemaphoreType.DMA,
        pltpu.SemaphoreType.DMA,
        pltpu.SemaphoreType.DMA,
        pltpu.SemaphoreType.DMA,
        pltpu.SemaphoreType.DMA,
        pltpu.SemaphoreType.DMA,
    ],
)
def _edge_dot_sc(table_hbm, psrc_hbm, pdst_hbm, nsrc_hbm, ndst_hbm,
                 out0_hbm, out1_hbm,
                 table_sh, sidx_a, didx_a, sidx_b, didx_b,
                 srow_a, drow_a, srow_b, drow_b,
                 part_a, part_b,
                 sem_ia, sem_ib, sem_sa, sem_da, sem_sb, sem_db,
                 sem_oa, sem_ob):
    cid = lax.axis_index("c")
    sid = lax.axis_index("s")
    hbase = sid * E_PER_W
    rbase = sid * (E_PER_W * L // 128)
    nchunk = jnp.where(sid == NS - 1, NCHUNK_LAST, NCHUNK_BASE)

    @pl.when(sid == 0)
    def _stage_table():
        pltpu.sync_copy(table_hbm, table_sh)

    plsc.subcore_barrier()

    def compute(srow, drow, part):
        @plsc.parallel_loop(0, CROWS, unroll=2)
        def _row(r):
            for j in range(128 // L):
                e = r * (128 // L) + j
                acc = srow[e, pl.ds(0, L)] * drow[e, pl.ds(0, L)]
                for k in range(1, D // L):
                    acc = acc + (srow[e, pl.ds(k * L, L)]
                                 * drow[e, pl.ds(k * L, L)])
                part[r, pl.ds(j * L, L)] = acc

    def run(esrc_hbm, edst_hbm, out_hbm):
        def issue_idx(g, sidx, didx, sem_i):
            off = hbase + g * CHUNK
            pltpu.async_copy(esrc_hbm.at[pl.ds(off, CHUNK)], sidx, sem_i)
            pltpu.async_copy(edst_hbm.at[pl.ds(off, CHUNK)], didx, sem_i)

        def wait_idx(sidx, didx, sem_i):
            pltpu.make_async_copy(
                esrc_hbm.at[pl.ds(hbase, CHUNK)], sidx, sem_i).wait()
            pltpu.make_async_copy(
                edst_hbm.at[pl.ds(hbase, CHUNK)], didx, sem_i).wait()

        def issue_rows_a(sidx, didx, srow, drow, sem_s, sem_d):
            pltpu.async_copy(table_sh.at[sidx], srow, sem_s)
            pltpu.async_copy(table_sh.at[didx], drow, sem_d)

        def wait_rows_a(sidx, didx, srow, drow, sem_s, sem_d):
            pltpu.make_async_copy(table_sh.at[sidx], srow, sem_s).wait()
            pltpu.make_async_copy(table_sh.at[didx], drow, sem_d).wait()

        issue_rows_b = issue_rows_a
        wait_rows_b = wait_rows_a

        def wait_out(part, sem_o):
            pltpu.make_async_copy(
                part, out_hbm.at[pl.ds(rbase, CROWS)], sem_o).wait()

        issue_idx(0, sidx_a, didx_a, sem_ia)
        issue_idx(1, sidx_b, didx_b, sem_ib)
        wait_idx(sidx_a, didx_a, sem_ia)
        issue_rows_a(sidx_a, didx_a, srow_a, drow_a, sem_sa, sem_da)

        def pair_body(i, carry):
            ga = 2 * i
            gb = ga + 1
            wait_idx(sidx_b, didx_b, sem_ib)
            issue_rows_b(sidx_b, didx_b, srow_b, drow_b, sem_sb, sem_db)

            wait_rows_a(sidx_a, didx_a, srow_a, drow_a, sem_sa, sem_da)

            @pl.when(ga + 2 < nchunk)
            def _idx_a():
                issue_idx(ga + 2, sidx_a, didx_a, sem_ia)

            @pl.when(i > 0)
            def _drain_a():
                wait_out(part_a, sem_oa)

            compute(srow_a, drow_a, part_a)
            pltpu.async_copy(
                part_a, out_hbm.at[pl.ds(rbase + ga * CROWS, CROWS)], sem_oa)

            wait_rows_b(sidx_b, didx_b, srow_b, drow_b, sem_sb, sem_db)

            @pl.when(gb + 2 < nchunk)
            def _idx_b():
                issue_idx(gb + 2, sidx_b, didx_b, sem_ib)

            @pl.when(i > 0)
            def _drain_b():
                wait_out(part_b, sem_ob)

            compute(srow_b, drow_b, part_b)
            pltpu.async_copy(
                part_b, out_hbm.at[pl.ds(rbase + gb * CROWS, CROWS)], sem_ob)

            @pl.when(ga + 2 < nchunk)
            def _rows_a():
                wait_idx(sidx_a, didx_a, sem_ia)
                issue_rows_a(sidx_a, didx_a, srow_a, drow_a, sem_sa, sem_da)

            return carry

        lax.fori_loop(0, nchunk // 2, pair_body, 0)
        wait_out(part_a, sem_oa)
        wait_out(part_b, sem_ob)

    @pl.when(cid == 0)
    def _core0():
        run(psrc_hbm, pdst_hbm, out0_hbm)

    @pl.when(cid == 1)
    def _core1():
        run(nsrc_hbm, ndst_hbm, out1_hbm)


_BLK = 5000
_NBLK = _ROWS_H // _BLK


def _loss_body(p_pos_ref, p_neg_ref, out_ref):
    pid = pl.program_id(0)
    gj = lax.broadcasted_iota(jnp.int32, (128, 8), 0) // L
    gg = lax.broadcasted_iota(jnp.int32, (128, 8), 1)
    m = (gj == gg).astype(jnp.float32)
    s0 = jnp.dot(p_pos_ref[...], m,
                 preferred_element_type=jnp.float32)
    s1 = jnp.dot(p_neg_ref[...], m,
                 preferred_element_type=jnp.float32)
    l0 = jnp.maximum(-s0, 0.0) + jnp.log1p(jnp.exp(-jnp.abs(s0)))
    l1 = jnp.maximum(s1, 0.0) + jnp.log1p(jnp.exp(-jnp.abs(s1)))
    bsum = jnp.sum(l0) + jnp.sum(l1)

    @pl.when(pid == 0)
    def _init():
        out_ref[0, 0] = 0.0

    out_ref[0, 0] += bsum

    @pl.when(pid == pl.num_programs(0) - 1)
    def _fini():
        out_ref[0, 0] = out_ref[0, 0] / E_TOT


_loss_tc = pl.pallas_call(
    _loss_body,
    grid=(_NBLK,),
    in_specs=[
        pl.BlockSpec((_BLK, 128), lambda i: (i, 0)),
        pl.BlockSpec((_BLK, 128), lambda i: (i, 0)),
    ],
    out_specs=pl.BlockSpec(
        (1, 1), lambda i: (0, 0), memory_space=pltpu.SMEM),
    out_shape=jax.ShapeDtypeStruct((1, 1), jnp.float32),
)


def kernel(block_outputs, pos_edge_index, neg_edge_index):
    pe = pos_edge_index.astype(jnp.int32)
    ne = neg_edge_index.astype(jnp.int32)
    p_pos, p_neg = _edge_dot_sc(block_outputs, pe[0], pe[1], ne[0], ne[1])
    loss = _loss_tc(p_pos, p_neg)
    return loss[0, 0]
